# baseline (speedup 1.0000x reference)
import functools

import jax
import jax.numpy as jnp
from jax.experimental import pallas as pl
from jax.experimental.pallas import tpu as pltpu
from jax.experimental.pallas import tpu_sc as plsc

D = 64
GATHER_W = 128
TC_BLOCK = 2048


def _sc_gather(table_pairs, idx_pair, T):
    mesh = plsc.VectorSubcoreMesh(core_axis_name="core", subcore_axis_name="subcore")

    @functools.partial(
        pl.kernel,
        out_type=jax.ShapeDtypeStruct((T, 2 * D), jnp.float32),
        mesh=mesh,
    )
    def gather_kernel(tbl_hbm, idx_hbm, out_hbm):
        def body(i_vmem, o_vmem):
            pltpu.sync_copy(tbl_hbm.at[i_vmem.at[0]], o_vmem)

        pltpu.emit_pipeline(
            body,
            grid=(T // GATHER_W,),
            in_specs=[pl.BlockSpec((1, GATHER_W), index_map=lambda i: (0, i))],
            out_specs=[pl.BlockSpec((GATHER_W, 2 * D), index_map=lambda i: (i, 0))],
            core_axis_name=("core", "subcore"),
            dimension_semantics=(pltpu.PARALLEL,),
        )(idx_hbm, out_hbm)

    return gather_kernel(table_pairs, idx_pair)


def _highway_body(emb_ref, par_ref, wt0, bt0, wg0, bg0, wt1, bt1, wg1, bg1,
                  out_ref):
    e = emb_ref[...]
    p = par_ref[...]
    h = jnp.where(p > 0, e[:, D:], e[:, :D])
    for (wt, bt, wg, bg) in ((wt0, bt0, wg0, bg0), (wt1, bt1, wg1, bg1)):
        t = jnp.maximum(
            jnp.dot(h, wt[...], preferred_element_type=jnp.float32) + bt[...], 0.0)
        g = jax.nn.sigmoid(
            jnp.dot(h, wg[...], preferred_element_type=jnp.float32) + bg[...])
        h = g * t + (1.0 - g) * h
    out_ref[...] = h


def _tc_highway(emb128, parity, weights):
    T = emb128.shape[0]
    wspec = pl.BlockSpec((D, D), lambda i: (0, 0))
    bspec = pl.BlockSpec((1, D), lambda i: (0, 0))
    return pl.pallas_call(
        _highway_body,
        grid=(T // TC_BLOCK,),
        in_specs=[pl.BlockSpec((TC_BLOCK, 2 * D), lambda i: (i, 0)),
                  pl.BlockSpec((TC_BLOCK, 1), lambda i: (i, 0)),
                  wspec, bspec, wspec, bspec, wspec, bspec, wspec, bspec],
        out_specs=pl.BlockSpec((TC_BLOCK, D), lambda i: (i, 0)),
        out_shape=jax.ShapeDtypeStruct((T, D), jnp.float32),
    )(emb128, parity, *weights)


def kernel(batch, table, Wt0, bt0, Wg0, bg0, Wt1, bt1, Wg1, bg1):
    B, L = batch.shape
    T = B * L
    V = table.shape[0]
    idx = batch.reshape(-1).astype(jnp.int32)
    table_pairs = table.reshape(V // 2, 2 * D)
    idx_pair = (idx >> 1).reshape(1, T)
    parity = (idx & 1).reshape(T, 1)
    emb128 = _sc_gather(table_pairs, idx_pair, T)
    out = _tc_highway(emb128, parity,
                      (Wt0, bt0.reshape(1, D), Wg0, bg0.reshape(1, D),
                       Wt1, bt1.reshape(1, D), Wg1, bg1.reshape(1, D)))
    return out.reshape(B, L, D)
